# Initial kernel scaffold; baseline (speedup 1.0000x reference)
import jax
import jax.numpy as jnp
from jax.experimental import pallas as pl


def kernel(object_feats, pairs, confidence, W, b):
    raise NotImplementedError("write your pallas kernel here")


# speedup vs baseline: 16.7500x; 16.7500x over previous
import dataclasses
import functools

import jax
import jax.numpy as jnp
from jax import lax
from jax.experimental import pallas as pl
from jax.experimental.pallas import tpu as pltpu
from jax.experimental.pallas import tpu_sc as plsc

N = 10000
E = 320000
D = 128
NC = 2
NS = 16
NW = NC * NS
WIN = 80
GROUPS = WIN // 16
EDGES_PER_W = E // NW
WINDOWS = EDGES_PER_W // WIN
N_PAD = 10240
ROWS_PER_S = N_PAD // NS


def _emb_mm_body(x_ref, w_ref, b_ref, o_ref):
    o_ref[...] = (
        jnp.dot(x_ref[...], w_ref[...].T, preferred_element_type=jnp.float32)
        + b_ref[...]
    )


def _emb(x, W, b):
    return pl.pallas_call(
        _emb_mm_body,
        out_shape=jax.ShapeDtypeStruct((N, D), jnp.float32),
    )(x, W, b.reshape(1, D))


def _sc_body(emb_hbm, sub_hbm, obj_hbm, conf_hbm, msg_hbm, den_hbm,
             acc_sh, den_v, idx_s, idx_o, cv, wv, A, B, S):
    c = lax.axis_index("c")
    s = lax.axis_index("s")
    wid = s * NC + c

    @pl.loop(0, WIN)
    def _(r):
        for k in range(D // 16):
            S[r, pl.ds(k * 16, 16)] = jnp.zeros((16,), jnp.float32)

    @pl.loop(0, N_PAD // 16)
    def _(r):
        den_v[pl.ds(pl.multiple_of(r * 16, 16), 16)] = jnp.zeros((16,), jnp.float32)

    @pl.loop(0, ROWS_PER_S // WIN)
    def _(k):
        pltpu.sync_copy(
            S, acc_sh.at[pl.ds(s * ROWS_PER_S + k * WIN, WIN)]
        )

    plsc.subcore_barrier()

    base = wid * EDGES_PER_W

    @pl.loop(0, WINDOWS)
    def _(win):
        off = base + win * WIN
        pltpu.sync_copy(sub_hbm.at[pl.ds(off, WIN)], idx_s)
        pltpu.sync_copy(obj_hbm.at[pl.ds(off, WIN)], idx_o)
        pltpu.sync_copy(conf_hbm.at[pl.ds(off, WIN)], cv)
        pltpu.sync_copy(emb_hbm.at[idx_s], A)
        pltpu.sync_copy(emb_hbm.at[idx_o], B)

        @pl.loop(0, GROUPS)
        def _(g):
            o16 = pl.multiple_of(g * 16, 16)
            w16 = jnp.exp(cv[pl.ds(o16, 16)] - 10.0)
            wv[...] = w16
            iv_s = idx_s[pl.ds(o16, 16)]
            iv_o = idx_o[pl.ds(o16, 16)]
            lane = jax.lax.iota(jnp.int32, 16)
            for i in range(16):
                e = g * 16 + i
                ws = plsc.load_gather(wv, [jnp.full((16,), i, jnp.int32)])
                for k in range(D // 16):
                    sl = pl.ds(k * 16, 16)
                    S[e, sl] = ws * A[e, sl] * B[e, sl]
                m = lane == i
                plsc.addupdate_scatter(den_v, [iv_s], w16, mask=m)
                plsc.addupdate_scatter(den_v, [iv_o], w16, mask=m)

        pltpu.sync_copy(S, acc_sh.at[idx_s], add=True)
        pltpu.sync_copy(S, acc_sh.at[idx_o], add=True)

    plsc.subcore_barrier()
    pltpu.sync_copy(
        acc_sh.at[pl.ds(s * ROWS_PER_S, ROWS_PER_S)],
        msg_hbm.at[c, pl.ds(s * ROWS_PER_S, ROWS_PER_S)],
    )
    pltpu.sync_copy(den_v, den_hbm.at[pl.ds(wid * N_PAD, N_PAD)])


def _sc_scatter(emb, sub, obj, conf):
    mesh = plsc.VectorSubcoreMesh(core_axis_name="c", subcore_axis_name="s")
    cp = pltpu.CompilerParams()
    if "needs_layout_passes" in pltpu.CompilerParams.__dataclass_fields__:
        cp = dataclasses.replace(cp, needs_layout_passes=False)
    kfn = pl.kernel(
        compiler_params=cp,
        out_type=(
            jax.ShapeDtypeStruct((NC, N_PAD, D), jnp.float32),
            jax.ShapeDtypeStruct((NW * N_PAD,), jnp.float32),
        ),
        mesh=mesh,
        scratch_types=[
            pltpu.VMEM_SHARED((N_PAD, D), jnp.float32),
            pltpu.VMEM((N_PAD,), jnp.float32),
            pltpu.VMEM((WIN,), jnp.int32),
            pltpu.VMEM((WIN,), jnp.int32),
            pltpu.VMEM((WIN,), jnp.float32),
            pltpu.VMEM((16,), jnp.float32),
            pltpu.VMEM((WIN, D), jnp.float32),
            pltpu.VMEM((WIN, D), jnp.float32),
            pltpu.VMEM((WIN, D), jnp.float32),
        ],
    )(_sc_body)
    return kfn(emb, sub, obj, conf)


def _combine_body(x_ref, p_ref, d_ref, o_ref):
    num = x_ref[...] + p_ref[0, :N, :] + p_ref[1, :N, :]
    den = 1.0 + jnp.sum(d_ref[...], axis=0)[:N]
    o_ref[...] = num / den[:, None]


def _combine(x, partials, dens):
    return pl.pallas_call(
        _combine_body,
        out_shape=jax.ShapeDtypeStruct((N, D), jnp.float32),
    )(x, partials, dens.reshape(NW, N_PAD))


@jax.jit
def kernel(object_feats, pairs, confidence, W, b):
    sub = jnp.asarray(pairs[:, 0], jnp.int32)
    obj = jnp.asarray(pairs[:, 1], jnp.int32)
    emb = _emb(object_feats, W, b)
    partials, dens = _sc_scatter(emb, sub, obj, confidence)
    new_feats = _combine(object_feats, partials, dens)
    return (new_feats, pairs, confidence)


# speedup vs baseline: 33.4198x; 1.9952x over previous
import dataclasses

import jax
import jax.numpy as jnp
from jax import lax
from jax.experimental import pallas as pl
from jax.experimental.pallas import tpu as pltpu
from jax.experimental.pallas import tpu_sc as plsc

N = 10000
E = 320000
D = 128
NC = 2
NS = 16
NW = NC * NS
WIN = 64
GROUPS = WIN // 16
W = 158
EDB = 3 * WIN
E_PAD = NW * W * WIN
N_PAD = 10112
ROWS_PER_S = N_PAD // NS


def _emb_mm_body(x_ref, w_ref, b_ref, of_ref):
    emb = (
        jnp.dot(x_ref[...], w_ref[...].T, preferred_element_type=jnp.float32)
        + b_ref[...]
    )
    of_ref[...] = emb


def _emb(x, W_, b):
    return pl.pallas_call(
        _emb_mm_body,
        out_shape=jax.ShapeDtypeStruct((N, D), jnp.float32),
    )(x, W_, b.reshape(1, D))


def _sc_body(embf_hbm, embp_hbm, ed_hbm, msg_hbm, den_hbm,
             acc_sh, den_v, wv, ed0, ed1, A0, A1, B0, B1,
             sis0, sio0, sis1, sio1,
             se0, se1, sgA0, sgA1, sgB0, sgB1, ss0, ss1):
    c = lax.axis_index("c")
    s = lax.axis_index("s")
    wid = s * NC + c

    eds = (ed0, ed1)
    As = (A0, A1)
    Bs = (B0, B1)
    siss = (sis0, sis1)
    sios = (sio0, sio1)
    ses = (se0, se1)
    sgAs = (sgA0, sgA1)
    sgBs = (sgB0, sgB1)
    sss = (ss0, ss1)

    @pl.loop(0, WIN)
    def _(r):
        for k in range(D // 16):
            A0[r, pl.ds(k * 16, 16)] = jnp.zeros((16,), jnp.float32)

    for k in range(ROWS_PER_S // WIN):
        pltpu.sync_copy(A0, acc_sh.at[pl.ds(s * ROWS_PER_S + k * WIN, WIN)])
    _rem = ROWS_PER_S % WIN
    pltpu.sync_copy(
        A0.at[pl.ds(0, _rem)],
        acc_sh.at[pl.ds(s * ROWS_PER_S + ROWS_PER_S - _rem, _rem)],
    )

    @pl.loop(0, N_PAD // 16)
    def _(r):
        den_v[pl.ds(pl.multiple_of(r * 16, 16), 16)] = jnp.zeros((16,), jnp.float32)

    plsc.subcore_barrier()

    base = wid * W

    pltpu.sync_copy(ed_hbm.at[pl.ds(base * EDB, EDB)], ed0)
    pltpu.async_copy(embf_hbm.at[ed0.at[pl.ds(0, WIN)]], A0, sgA0)
    pltpu.async_copy(embp_hbm.at[ed0.at[pl.ds(WIN, WIN)]], B0, sgB0)
    pltpu.async_copy(ed_hbm.at[pl.ds((base + 1) * EDB, EDB)], ed1, se1)

    @pl.loop(0, W, step=2)
    def _(j):
        for p in (0, 1):
            jj = j + p
            ed, A, B = eds[p], As[p], Bs[p]
            edn, An, Bn = eds[1 - p], As[1 - p], Bs[1 - p]

            def _prefetch():
                pltpu.make_async_copy(
                    ed_hbm.at[pl.ds((base + jj + 1) * EDB, EDB)],
                    edn, ses[1 - p]).wait()
                pltpu.async_copy(
                    embf_hbm.at[edn.at[pl.ds(0, WIN)]], An, sgAs[1 - p])
                pltpu.async_copy(
                    embp_hbm.at[edn.at[pl.ds(WIN, WIN)]], Bn, sgBs[1 - p])

            def _wait_prev_scatters():
                pltpu.make_async_copy(
                    An, acc_sh.at[siss[1 - p]], sss[1 - p]).wait()
                pltpu.make_async_copy(
                    An, acc_sh.at[sios[1 - p]], sss[1 - p]).wait()

            if p == 0:
                @pl.when(jj >= 1)
                def _():
                    _wait_prev_scatters()

                _prefetch()
            else:
                @pl.when(jj < W - 1)
                def _():
                    _wait_prev_scatters()
                    _prefetch()

            pltpu.make_async_copy(
                embf_hbm.at[ed.at[pl.ds(0, WIN)]], A, sgAs[p]).wait()
            pltpu.make_async_copy(
                embp_hbm.at[ed.at[pl.ds(WIN, WIN)]], B, sgBs[p]).wait()

            @pl.loop(0, GROUPS)
            def _(g):
                o16 = pl.multiple_of(g * 16, 16)
                cb = ed[pl.ds(pl.multiple_of(2 * WIN + g * 16, 16), 16)]
                w16 = jnp.exp(plsc.bitcast(cb, jnp.float32) - 10.0)
                wv[...] = w16
                iv_s = ed[pl.ds(o16, 16)]
                iv_o = ed[pl.ds(pl.multiple_of(WIN + g * 16, 16), 16)]
                lane = lax.iota(jnp.int32, 16)
                for i in range(16):
                    e = g * 16 + i
                    ws = plsc.load_gather(wv, [jnp.full((16,), i, jnp.int32)])
                    for ch in range(8):
                        sl = pl.ds(ch * 16, 16)
                        A[e, sl] = ws * A[e, sl] * B[e, sl]
                    m = lane == i
                    plsc.addupdate_scatter(den_v, [iv_s], w16, mask=m)
                    plsc.addupdate_scatter(den_v, [iv_o], w16, mask=m)

            for k in range(GROUPS):
                siss[p][pl.ds(k * 16, 16)] = ed[pl.ds(k * 16, 16)]
                sios[p][pl.ds(k * 16, 16)] = ed[
                    pl.ds(pl.multiple_of(WIN + k * 16, 16), 16)]

            pltpu.async_copy(A, acc_sh.at[siss[p]], sss[p], add=True)
            pltpu.async_copy(A, acc_sh.at[sios[p]], sss[p], add=True)

            @pl.when(jj < W - 2)
            def _():
                pltpu.async_copy(
                    ed_hbm.at[pl.ds((base + jj + 2) * EDB, EDB)], ed, ses[p])

    for p in (0, 1):
        pltpu.make_async_copy(As[p], acc_sh.at[siss[p]], sss[p]).wait()
        pltpu.make_async_copy(As[p], acc_sh.at[sios[p]], sss[p]).wait()

    plsc.subcore_barrier()
    pltpu.sync_copy(
        acc_sh.at[pl.ds(s * ROWS_PER_S, ROWS_PER_S)],
        msg_hbm.at[c, pl.ds(s * ROWS_PER_S, ROWS_PER_S)],
    )
    pltpu.sync_copy(den_v, den_hbm.at[pl.ds(wid * N_PAD, N_PAD)])


def _sc_scatter(embf, embp, edata):
    mesh = plsc.VectorSubcoreMesh(core_axis_name="c", subcore_axis_name="s")
    cp = pltpu.CompilerParams()
    if "needs_layout_passes" in pltpu.CompilerParams.__dataclass_fields__:
        cp = dataclasses.replace(cp, needs_layout_passes=False)
    kfn = pl.kernel(
        compiler_params=cp,
        out_type=(
            jax.ShapeDtypeStruct((NC, N_PAD, D), jnp.float32),
            jax.ShapeDtypeStruct((NW * N_PAD,), jnp.float32),
        ),
        mesh=mesh,
        scratch_types=[
            pltpu.VMEM_SHARED((N_PAD, D), jnp.float32),
            pltpu.VMEM((N_PAD,), jnp.float32),
            pltpu.VMEM((16,), jnp.float32),
            pltpu.VMEM((EDB,), jnp.int32),
            pltpu.VMEM((EDB,), jnp.int32),
            pltpu.VMEM((WIN, D), jnp.float32),
            pltpu.VMEM((WIN, D), jnp.float32),
            pltpu.VMEM((WIN, D), jnp.float32),
            pltpu.VMEM((WIN, D), jnp.float32),
            pltpu.VMEM((WIN,), jnp.int32),
            pltpu.VMEM((WIN,), jnp.int32),
            pltpu.VMEM((WIN,), jnp.int32),
            pltpu.VMEM((WIN,), jnp.int32),
            pltpu.SemaphoreType.DMA,
            pltpu.SemaphoreType.DMA,
            pltpu.SemaphoreType.DMA,
            pltpu.SemaphoreType.DMA,
            pltpu.SemaphoreType.DMA,
            pltpu.SemaphoreType.DMA,
            pltpu.SemaphoreType.DMA,
            pltpu.SemaphoreType.DMA,
        ],
    )(_sc_body)
    return kfn(embf, embp, edata)


def _combine_body(x_ref, p_ref, d_ref, o_ref):
    num = x_ref[...] + p_ref[0, :N, :] + p_ref[1, :N, :]
    den = 1.0 + jnp.sum(d_ref[...], axis=0)[:N]
    o_ref[...] = num / den[:, None]


def _combine(x, partials, dens):
    return pl.pallas_call(
        _combine_body,
        out_shape=jax.ShapeDtypeStruct((N, D), jnp.float32),
    )(x, partials, dens.reshape(NW, N_PAD))


@jax.jit
def kernel(object_feats, pairs, confidence, W_, b):
    sub = jnp.asarray(pairs[:, 0], jnp.int32)
    obj = jnp.asarray(pairs[:, 1], jnp.int32)
    npad = E_PAD - E
    iota = jnp.arange(npad, dtype=jnp.int32)
    sub_p = jnp.concatenate([sub, iota % N])
    obj_p = jnp.concatenate([obj, (iota * 7 + 13) % N])
    conf_p = jnp.concatenate(
        [confidence, jnp.full((npad,), -100.0, jnp.float32)])
    edata = jnp.stack(
        [
            sub_p.reshape(-1, WIN),
            obj_p.reshape(-1, WIN),
            jax.lax.bitcast_convert_type(conf_p, jnp.int32).reshape(-1, WIN),
        ],
        axis=1,
    ).reshape(-1)
    embf = _emb(object_feats, W_, b)
    partials, dens = _sc_scatter(embf, embf, edata)
    new_feats = _combine(object_feats, partials, dens)
    return (new_feats, pairs, confidence)
